# baseline (speedup 1.0000x reference)
import functools

import jax
import jax.numpy as jnp
from jax import lax
from jax.experimental import pallas as pl
from jax.experimental.pallas import tpu as pltpu
from jax.experimental.pallas import tpu_sc as plsc

B = 4
S = 2048
D = 128
FLAT = B * S
NC, NS, L = 2, 16, 16
NW = NC * NS
N = FLAT // NW
HALO = 8
CH = N + L
NIDX = CH // L
NIDXP = 24
NGRP = N // L
WPS = S // N


def _sc_body(xp_ref, table_ref, blanks_ref, out_ref,
             idx_v, rows_v, out_v, isb_v, m_v, c1_v, c2_v, c3_v, blk_v, sem):
    wid = lax.axis_index("s") * NC + lax.axis_index("c")
    base = wid * N

    pltpu.sync_copy(xp_ref.at[pl.ds(wid * L, NIDXP)], idx_v)
    pltpu.sync_copy(blanks_ref, blk_v)

    descs = [
        pltpu.async_copy(table_ref.at[idx_v[j]], rows_v.at[pl.ds(j * L, L)], sem)
        for j in range(NIDX)
    ]

    b0 = blk_v[0]
    b1 = blk_v[1]
    b2 = blk_v[2]
    b3 = blk_v[3]
    for j in range(NIDX):
        v = idx_v[j]
        hit = (v == b0) | (v == b1) | (v == b2) | (v == b3)
        isb_v[pl.ds(j * L, L)] = jnp.where(hit, 1.0, 0.0)
    isb_v[pl.ds(NIDX * L, L)] = jnp.zeros((L,), jnp.float32)

    for j in range(NIDX):
        m_v[pl.ds(j * L, L)] = (
            isb_v[pl.ds(j * L + 1, L)] * (1.0 - isb_v[pl.ds(j * L, L)])
        )

    lane = lax.iota(jnp.int32, L)
    halo_keep = jnp.where(lane < HALO, 0.0, 1.0)

    @pl.when(wid % WPS == 0)
    def _zero_halo():
        m_v[pl.ds(0, L)] = m_v[pl.ds(0, L)] * halo_keep

    for g in range(NGRP):
        j0 = HALO + g * L
        m1 = m_v[pl.ds(j0 - 1, L)]
        m2 = m_v[pl.ds(j0 - 2, L)]
        m3 = m_v[pl.ds(j0 - 3, L)]
        c1_v[pl.ds(g * L, L)] = m1 + m2 + m3
        c2_v[pl.ds(g * L, L)] = m2 + m3 + m2 * m3
        c3_v[pl.ds(g * L, L)] = m3

    for d in descs:
        d.wait()

    ND = D // L

    def _row(j):
        return tuple(rows_v[j, pl.ds(dd * L, L)] for dd in range(ND))

    def gstep(g, carry):
        r1, r2, r3 = carry
        j0 = HALO + g * L
        c1g = c1_v[pl.ds(g * L, L)]
        c2g = c2_v[pl.ds(g * L, L)]
        c3g = c3_v[pl.ds(g * L, L)]
        for t2 in range(L):
            j = j0 + t2
            lanes = jnp.full((L,), t2, jnp.int32)
            w1 = c1g.at[lanes].get(mode="promise_in_bounds")
            w2 = c2g.at[lanes].get(mode="promise_in_bounds")
            w3 = c3g.at[lanes].get(mode="promise_in_bounds")
            e0 = _row(j)
            for dd in range(ND):
                out_v[g * L + t2, pl.ds(dd * L, L)] = (
                    e0[dd] + w1 * r1[dd] + w2 * r2[dd] + w3 * r3[dd]
                )
            r3, r2, r1 = r2, r1, e0
        return (r1, r2, r3)

    lax.fori_loop(0, NGRP, gstep,
                  (_row(HALO - 1), _row(HALO - 2), _row(HALO - 3)))

    pltpu.sync_copy(out_v, out_ref.at[pl.ds(base, N)])


@jax.jit
def _blank_embedding(xp, table, blanks):
    mesh = plsc.VectorSubcoreMesh(core_axis_name="c", subcore_axis_name="s")
    run = functools.partial(
        pl.kernel,
        out_type=jax.ShapeDtypeStruct((FLAT, D), jnp.float32),
        mesh=mesh,
        scratch_types=[
            pltpu.VMEM((NIDXP, L), jnp.int32),
            pltpu.VMEM((CH, D), jnp.float32),
            pltpu.VMEM((N, D), jnp.float32),
            pltpu.VMEM((CH + L, ), jnp.float32),
            pltpu.VMEM((CH,), jnp.float32),
            pltpu.VMEM((N,), jnp.float32),
            pltpu.VMEM((N,), jnp.float32),
            pltpu.VMEM((N,), jnp.float32),
            pltpu.VMEM((8, L), jnp.int32),
            pltpu.SemaphoreType.DMA,
        ],
    )(_sc_body)
    return run(xp, table, blanks)


def kernel(x, table, blank_ids):
    xf = x.reshape(-1).astype(jnp.int32)
    tail = L - HALO + (NIDXP - NIDX) * L
    xp = jnp.concatenate([
        jnp.zeros((HALO,), jnp.int32), xf, jnp.zeros((tail,), jnp.int32)
    ])
    xp2 = xp.reshape(-1, L)
    blanks = jnp.tile(jnp.tile(blank_ids.astype(jnp.int32), 2)[:, None],
                      (1, L))
    out = _blank_embedding(xp2, table, blanks)
    return out.reshape(B, S, D)


# speedup vs baseline: 1.0160x; 1.0160x over previous
import functools

import jax
import jax.numpy as jnp
from jax import lax
from jax.experimental import pallas as pl
from jax.experimental.pallas import tpu as pltpu
from jax.experimental.pallas import tpu_sc as plsc

B = 4
S = 2048
D = 128
FLAT = B * S
NC, NS, L = 2, 16, 16
NW = NC * NS
N = FLAT // NW
HALO = 8
CH = N + L
NIDX = CH // L
NIDXP = 24
NGRP = N // L
WPS = S // N


def _sc_body(xp_ref, table_ref, blanks_ref, out_ref,
             idx_v, rows_v, out_v, isb_v, m_v, c1_v, c2_v, c3_v, blk_v, sem):
    wid = lax.axis_index("s") * NC + lax.axis_index("c")
    base = wid * N

    pltpu.sync_copy(xp_ref.at[pl.ds(wid * L, NIDXP)], idx_v)
    pltpu.sync_copy(blanks_ref, blk_v)

    descs = [
        pltpu.async_copy(table_ref.at[idx_v[j]], rows_v.at[pl.ds(j * L, L)], sem)
        for j in range(NIDX)
    ]

    b0 = blk_v[0]
    b1 = blk_v[1]
    b2 = blk_v[2]
    b3 = blk_v[3]
    for j in range(NIDX):
        v = idx_v[j]
        hit = (v == b0) | (v == b1) | (v == b2) | (v == b3)
        isb_v[pl.ds(j * L, L)] = jnp.where(hit, 1.0, 0.0)
    isb_v[pl.ds(NIDX * L, L)] = jnp.zeros((L,), jnp.float32)

    for j in range(NIDX):
        m_v[pl.ds(j * L, L)] = (
            isb_v[pl.ds(j * L + 1, L)] * (1.0 - isb_v[pl.ds(j * L, L)])
        )

    lane = lax.iota(jnp.int32, L)
    halo_keep = jnp.where(lane < HALO, 0.0, 1.0)

    @pl.when(wid % WPS == 0)
    def _zero_halo():
        m_v[pl.ds(0, L)] = m_v[pl.ds(0, L)] * halo_keep

    for g in range(NGRP):
        j0 = HALO + g * L
        m1 = m_v[pl.ds(j0 - 1, L)]
        m2 = m_v[pl.ds(j0 - 2, L)]
        m3 = m_v[pl.ds(j0 - 3, L)]
        c1_v[pl.ds(g * L, L)] = m1 + m2 + m3
        c2_v[pl.ds(g * L, L)] = m2 + m3 + m2 * m3
        c3_v[pl.ds(g * L, L)] = m3

    for d in descs:
        d.wait()

    ND = D // L

    def _row(j):
        return tuple(rows_v[j, pl.ds(dd * L, L)] for dd in range(ND))

    def gstep(g, carry):
        r1, r2, r3 = carry
        j0 = HALO + g * L
        c1g = c1_v[pl.ds(g * L, L)]
        c2g = c2_v[pl.ds(g * L, L)]
        c3g = c3_v[pl.ds(g * L, L)]
        for t2 in range(L):
            j = j0 + t2
            lanes = jnp.full((L,), t2, jnp.int32)
            w1 = c1g.at[lanes].get(mode="promise_in_bounds")
            w2 = c2g.at[lanes].get(mode="promise_in_bounds")
            w3 = c3g.at[lanes].get(mode="promise_in_bounds")
            e0 = _row(j)
            for dd in range(ND):
                out_v[g * L + t2, pl.ds(dd * L, L)] = (
                    e0[dd] + w1 * r1[dd] + w2 * r2[dd] + w3 * r3[dd]
                )
            r3, r2, r1 = r2, r1, e0
        return (r1, r2, r3)

    lax.fori_loop(0, NGRP, gstep,
                  (_row(HALO - 1), _row(HALO - 2), _row(HALO - 3)))

    pltpu.sync_copy(out_v, out_ref.at[pl.ds(base, N)])


@jax.jit
def _blank_embedding(xp, table, blanks):
    mesh = plsc.VectorSubcoreMesh(core_axis_name="c", subcore_axis_name="s")
    run = functools.partial(
        pl.kernel,
        out_type=jax.ShapeDtypeStruct((FLAT, D), jnp.float32),
        mesh=mesh,
        compiler_params=pltpu.CompilerParams(use_tc_tiling_on_sc=False),
        scratch_types=[
            pltpu.VMEM((NIDXP, L), jnp.int32),
            pltpu.VMEM((CH, D), jnp.float32),
            pltpu.VMEM((N, D), jnp.float32),
            pltpu.VMEM((CH + L, ), jnp.float32),
            pltpu.VMEM((CH,), jnp.float32),
            pltpu.VMEM((N,), jnp.float32),
            pltpu.VMEM((N,), jnp.float32),
            pltpu.VMEM((N,), jnp.float32),
            pltpu.VMEM((8, L), jnp.int32),
            pltpu.SemaphoreType.DMA,
        ],
    )(_sc_body)
    return run(xp, table, blanks)


def kernel(x, table, blank_ids):
    xf = x.reshape(-1).astype(jnp.int32)
    tail = L - HALO + (NIDXP - NIDX) * L
    xp = jnp.concatenate([
        jnp.zeros((HALO,), jnp.int32), xf, jnp.zeros((tail,), jnp.int32)
    ])
    xp2 = xp.reshape(-1, L)
    blanks = jnp.tile(jnp.tile(blank_ids.astype(jnp.int32), 2)[:, None],
                      (1, L))
    out = _blank_embedding(xp2, table, blanks)
    return out.reshape(B, S, D)
